# baseline (speedup 1.0000x reference)
import functools

import jax
import jax.numpy as jnp
import numpy as np
from jax.experimental import pallas as pl
from jax.experimental.pallas import tpu as pltpu

_VMEM_LIMIT = 100 * 1024 * 1024
_BLK = 64
_NEG = -1e30


def _stage1_band_indicator():
    dw = np.arange(3).reshape(3, 1, 1, 1)
    wi = np.arange(64).reshape(1, 64, 1, 1)
    p = np.arange(2).reshape(1, 1, 2, 1)
    u = np.arange(1, 33).reshape(1, 1, 1, 32)
    return (wi == 2 * u + p + dw - 3).astype(np.float32)


def _stageN_band_indicator(W):
    dw = np.arange(3).reshape(3, 1, 1, 1)
    ui = np.arange(W + 2).reshape(1, W + 2, 1, 1)
    p = np.arange(2).reshape(1, 1, 2, 1)
    u = np.arange(1, W // 2 + 1).reshape(1, 1, 1, W // 2)
    return (ui == 2 * u + p + dw - 2).astype(np.float32)


_I1 = _stage1_band_indicator()
_J2 = _stageN_band_indicator(32)
_J3 = _stageN_band_indicator(16)


def _band1(w1):
    r = jnp.einsum("oihd,dwpu->ihwpou", w1, _I1)
    r = jnp.pad(r, ((0, 0),) * 5 + ((1, 1),))
    return r.reshape(3 * 3 * 64, 2 * 16 * 34).astype(jnp.bfloat16)


def _bandN(w, ind, W):
    Cout, Cin = w.shape[0], w.shape[1]
    r = jnp.einsum("oihd,dzpu->hizpou", w, ind)
    r = jnp.pad(r, ((0, 0),) * 5 + ((1, 1),))
    return r.reshape(3 * Cin * (W + 2),
                     2 * Cout * (W // 2 + 2)).astype(jnp.bfloat16)


def _bias_ext(b, U):
    core = jnp.broadcast_to(b.reshape(1, -1, 1), (2, b.shape[0], U - 2))
    return jnp.pad(core, ((0, 0), (0, 0), (1, 1)),
                   constant_values=_NEG).reshape(1, -1).astype(jnp.float32)


def _conv_pool(x3_scr, wb_ref, be_ref, *, BLK, H, L, Nh):
    xc = jnp.concatenate(
        [x3_scr[:, dh:dh + H, :].reshape(BLK * H, L) for dh in range(3)],
        axis=1)
    acc = jnp.dot(xc, wb_ref[...], preferred_element_type=jnp.float32)
    y = jnp.maximum(acc + be_ref[...], 0.0)
    y2 = y.reshape(BLK * H // 2, 2, 2 * Nh)
    yr = jnp.maximum(y2[:, 0], y2[:, 1])
    return jnp.maximum(yr[:, :Nh], yr[:, Nh:])


def _fused_kernel(x_ref, wb1_ref, be1_ref, wb2_ref, be2_ref, wb3_ref,
                  be3_ref, wf_ref, bfc_ref, o_ref, xs1, xs2, xs3, *, BLK):
    xs1[...] = jnp.zeros_like(xs1)
    xs1[:, :, 1:65, :] = x_ref[...].astype(jnp.bfloat16)
    xc1 = jnp.concatenate(
        [xs1[:, ci, dh:dh + 64, :].reshape(BLK * 64, 64)
         for ci in range(3) for dh in range(3)], axis=1)
    acc = jnp.dot(xc1, wb1_ref[...], preferred_element_type=jnp.float32)
    y = jnp.maximum(acc + be1_ref[...], 0.0)
    y2 = y.reshape(BLK * 32, 2, 1088)
    yr = jnp.maximum(y2[:, 0], y2[:, 1])
    p1 = jnp.maximum(yr[:, :544], yr[:, 544:])

    xs2[...] = jnp.zeros_like(xs2)
    xs2[:, 1:33, :] = p1.reshape(BLK, 32, 544).astype(jnp.bfloat16)
    p2 = _conv_pool(xs2, wb2_ref, be2_ref, BLK=BLK, H=32, L=544, Nh=576)

    xs3[...] = jnp.zeros_like(xs3)
    xs3[:, 1:17, :] = p2.reshape(BLK, 16, 576).astype(jnp.bfloat16)
    p3 = _conv_pool(xs3, wb3_ref, be3_ref, BLK=BLK, H=16, L=576, Nh=640)

    r = p3.reshape(BLK, 8, 640)
    l0 = jnp.sum(jnp.sum(r * wf_ref[0], axis=2), axis=1, keepdims=True)
    l1 = jnp.sum(jnp.sum(r * wf_ref[1], axis=2), axis=1, keepdims=True)
    lane = jax.lax.broadcasted_iota(jnp.int32, (BLK, 2), 1)
    o_ref[...] = jnp.where(lane == 0, l0, l1) + bfc_ref[...]


def kernel(x, w1, b1, w2, b2, w3, b3, wfc, bfc):
    B = x.shape[0]
    BLK = _BLK
    wb1 = _band1(w1)
    wb2 = _bandN(w2, _J2, 32)
    wb3 = _bandN(w3, _J3, 16)
    be1 = _bias_ext(b1, 34)
    be2 = _bias_ext(b2, 18)
    be3 = _bias_ext(b3, 10)
    wf = jnp.pad(wfc.reshape(64, 8, 8, 2).transpose(3, 1, 0, 2),
                 ((0, 0), (0, 0), (0, 0), (1, 1))).reshape(2, 8, 640)
    wf = wf.astype(jnp.float32)
    bfc_p = bfc.reshape(1, 2).astype(jnp.float32)

    kernel_fn = functools.partial(_fused_kernel, BLK=BLK)
    out = pl.pallas_call(
        kernel_fn,
        out_shape=jax.ShapeDtypeStruct((B, 2), jnp.float32),
        grid=(B // BLK,),
        in_specs=[
            pl.BlockSpec((BLK, 3, 64, 64), lambda i: (i, 0, 0, 0)),
            pl.BlockSpec((576, 1088), lambda i: (0, 0)),
            pl.BlockSpec((1, 1088), lambda i: (0, 0)),
            pl.BlockSpec((1632, 1152), lambda i: (0, 0)),
            pl.BlockSpec((1, 1152), lambda i: (0, 0)),
            pl.BlockSpec((1728, 1280), lambda i: (0, 0)),
            pl.BlockSpec((1, 1280), lambda i: (0, 0)),
            pl.BlockSpec((2, 8, 640), lambda i: (0, 0, 0)),
            pl.BlockSpec((1, 2), lambda i: (0, 0)),
        ],
        out_specs=pl.BlockSpec((BLK, 2), lambda i: (i, 0)),
        scratch_shapes=[
            pltpu.VMEM((BLK, 3, 66, 64), jnp.bfloat16),
            pltpu.VMEM((BLK, 34, 544), jnp.bfloat16),
            pltpu.VMEM((BLK, 18, 576), jnp.bfloat16),
        ],
        compiler_params=pltpu.CompilerParams(
            dimension_semantics=("parallel",),
            vmem_limit_bytes=_VMEM_LIMIT),
    )(x, wb1, be1, wb2, be2, wb3, be3, wf, bfc_p)
    return out


# speedup vs baseline: 1.5755x; 1.5755x over previous
import functools

import jax
import jax.numpy as jnp
import numpy as np
from jax.experimental import pallas as pl
from jax.experimental.pallas import tpu as pltpu

_VMEM_LIMIT = 100 * 1024 * 1024
_BLK = 32


def _band_indicator(Win):
    dw = np.arange(3).reshape(3, 1, 1, 1)
    wi = np.arange(Win).reshape(1, Win, 1, 1)
    p = np.arange(2).reshape(1, 1, 2, 1)
    wo = np.arange(Win // 2).reshape(1, 1, 1, Win // 2)
    return (wi == 2 * wo + p + dw - 1).astype(np.float32)


_J1 = _band_indicator(64)
_J2 = _band_indicator(32)
_J3 = _band_indicator(16)


def _band1(w1):
    r = jnp.einsum("oihd,dwpu->ihwpou", w1, _J1)
    return r.reshape(3 * 3 * 64, 2 * 16 * 32).astype(jnp.bfloat16)


def _bandN(w, ind, Win):
    Cout, Cin = w.shape[0], w.shape[1]
    r = jnp.einsum("oihd,dzpu->hizpou", w, ind)
    return r.reshape(3 * Cin * Win, Cout * Win).astype(jnp.bfloat16)


def _bias_ext(b, Wout):
    return jnp.broadcast_to(b.reshape(1, -1, 1),
                            (2, b.shape[0], Wout)).reshape(1, -1)


def _conv_pool(x3_scr, wb_ref, be_ref, *, BLK, H, L, Nh):
    xc = jnp.concatenate(
        [x3_scr[:, dh:dh + H, :].reshape(BLK * H, L) for dh in range(3)],
        axis=1)
    acc = jnp.dot(xc, wb_ref[...], preferred_element_type=jnp.float32)
    y = jnp.maximum(acc + be_ref[...], 0.0)
    y2 = y.reshape(BLK * H // 2, 2, 2 * Nh)
    yr = jnp.maximum(y2[:, 0], y2[:, 1])
    return jnp.maximum(yr[:, :Nh], yr[:, Nh:])


def _fused_kernel(x_ref, wb1_ref, be1_ref, wb2_ref, be2_ref, wb3_ref,
                  be3_ref, wf_ref, bfc_ref, o_ref, xs1, xs2, xs3, *, BLK):
    xs1[...] = jnp.zeros_like(xs1)
    xs1[:, :, 1:65, :] = x_ref[...].astype(jnp.bfloat16)
    xc1 = jnp.concatenate(
        [xs1[:, ci, dh:dh + 64, :].reshape(BLK * 64, 64)
         for ci in range(3) for dh in range(3)], axis=1)
    acc = jnp.dot(xc1, wb1_ref[...], preferred_element_type=jnp.float32)
    y = jnp.maximum(acc + be1_ref[...], 0.0)
    y2 = y.reshape(BLK * 32, 2, 1024)
    yr = jnp.maximum(y2[:, 0], y2[:, 1])
    p1 = jnp.maximum(yr[:, :512], yr[:, 512:])

    xs2[...] = jnp.zeros_like(xs2)
    xs2[:, 1:33, :] = p1.reshape(BLK, 32, 512).astype(jnp.bfloat16)
    p2 = _conv_pool(xs2, wb2_ref, be2_ref, BLK=BLK, H=32, L=512, Nh=512)

    xs3[...] = jnp.zeros_like(xs3)
    xs3[:, 1:17, :] = p2.reshape(BLK, 16, 512).astype(jnp.bfloat16)
    p3 = _conv_pool(xs3, wb3_ref, be3_ref, BLK=BLK, H=16, L=512, Nh=512)

    r = p3.reshape(BLK, 8, 512)
    l0 = jnp.sum(jnp.sum(r * wf_ref[0], axis=2), axis=1, keepdims=True)
    l1 = jnp.sum(jnp.sum(r * wf_ref[1], axis=2), axis=1, keepdims=True)
    lane = jax.lax.broadcasted_iota(jnp.int32, (BLK, 2), 1)
    o_ref[...] = jnp.where(lane == 0, l0, l1) + bfc_ref[...]


def kernel(x, w1, b1, w2, b2, w3, b3, wfc, bfc):
    B = x.shape[0]
    BLK = _BLK
    wb1 = _band1(w1)
    wb2 = _bandN(w2, _J2, 32)
    wb3 = _bandN(w3, _J3, 16)
    be1 = _bias_ext(b1, 32)
    be2 = _bias_ext(b2, 16)
    be3 = _bias_ext(b3, 8)
    wf = wfc.reshape(64, 8, 8, 2).transpose(3, 1, 0, 2).reshape(2, 8, 512)
    wf = wf.astype(jnp.float32)
    bfc_p = bfc.reshape(1, 2).astype(jnp.float32)

    kernel_fn = functools.partial(_fused_kernel, BLK=BLK)
    out = pl.pallas_call(
        kernel_fn,
        out_shape=jax.ShapeDtypeStruct((B, 2), jnp.float32),
        grid=(B // BLK,),
        in_specs=[
            pl.BlockSpec((BLK, 3, 64, 64), lambda i: (i, 0, 0, 0)),
            pl.BlockSpec((576, 1024), lambda i: (0, 0)),
            pl.BlockSpec((1, 1024), lambda i: (0, 0)),
            pl.BlockSpec((1536, 1024), lambda i: (0, 0)),
            pl.BlockSpec((1, 1024), lambda i: (0, 0)),
            pl.BlockSpec((1536, 1024), lambda i: (0, 0)),
            pl.BlockSpec((1, 1024), lambda i: (0, 0)),
            pl.BlockSpec((2, 8, 512), lambda i: (0, 0, 0)),
            pl.BlockSpec((1, 2), lambda i: (0, 0)),
        ],
        out_specs=pl.BlockSpec((BLK, 2), lambda i: (i, 0)),
        scratch_shapes=[
            pltpu.VMEM((BLK, 3, 66, 64), jnp.bfloat16),
            pltpu.VMEM((BLK, 34, 512), jnp.bfloat16),
            pltpu.VMEM((BLK, 18, 512), jnp.bfloat16),
        ],
        compiler_params=pltpu.CompilerParams(
            dimension_semantics=("parallel",),
            vmem_limit_bytes=_VMEM_LIMIT),
    )(x, wb1, be1, wb2, be2, wb3, be3, wf, bfc_p)
    return out


# speedup vs baseline: 1.5767x; 1.0008x over previous
import functools

import jax
import jax.numpy as jnp
import numpy as np
from jax.experimental import pallas as pl
from jax.experimental.pallas import tpu as pltpu

_VMEM_LIMIT = 100 * 1024 * 1024
_BLK = 16


def _band_indicator(Win):
    dw = np.arange(3).reshape(3, 1, 1, 1)
    wi = np.arange(Win).reshape(1, Win, 1, 1)
    p = np.arange(2).reshape(1, 1, 2, 1)
    wo = np.arange(Win // 2).reshape(1, 1, 1, Win // 2)
    return (wi == 2 * wo + p + dw - 1).astype(np.float32)


_J1 = _band_indicator(64)
_J2 = _band_indicator(32)
_J3 = _band_indicator(16)


def _band1(w1):
    r = jnp.einsum("oihd,dwpu->ihwpou", w1, _J1)
    return r.reshape(3 * 3 * 64, 2 * 16 * 32).astype(jnp.bfloat16)


def _bandN(w, ind, Win):
    Cout, Cin = w.shape[0], w.shape[1]
    r = jnp.einsum("oihd,dzpu->hizpou", w, ind)
    return r.reshape(3 * Cin * Win, Cout * Win).astype(jnp.bfloat16)


def _bias_ext(b, Wout):
    return jnp.broadcast_to(b.reshape(1, -1, 1),
                            (2, b.shape[0], Wout)).reshape(1, -1)


def _conv_pool(x3_scr, wb_ref, be_ref, *, BLK, H, L, Nh):
    xc = jnp.concatenate(
        [x3_scr[:, dh:dh + H, :].reshape(BLK * H, L) for dh in range(3)],
        axis=1)
    acc = jnp.dot(xc, wb_ref[...], preferred_element_type=jnp.float32)
    y = jnp.maximum(acc + be_ref[...], 0.0)
    y2 = y.reshape(BLK * H // 2, 2, 2 * Nh)
    yr = jnp.maximum(y2[:, 0], y2[:, 1])
    return jnp.maximum(yr[:, :Nh], yr[:, Nh:])


def _fused_kernel(x_ref, wb1_ref, be1_ref, wb2_ref, be2_ref, wb3_ref,
                  be3_ref, wf_ref, bfc_ref, o_ref, xs1, xs2, xs3, *, BLK):
    xs1[...] = jnp.zeros_like(xs1)
    xs1[:, :, 1:65, :] = x_ref[...].astype(jnp.bfloat16)
    xc1 = jnp.concatenate(
        [xs1[:, ci, dh:dh + 64, :].reshape(BLK * 64, 64)
         for ci in range(3) for dh in range(3)], axis=1)
    acc = jnp.dot(xc1, wb1_ref[...], preferred_element_type=jnp.float32)
    y = jnp.maximum(acc + be1_ref[...], 0.0)
    y2 = y.reshape(BLK * 32, 2, 1024)
    yr = jnp.maximum(y2[:, 0], y2[:, 1])
    p1 = jnp.maximum(yr[:, :512], yr[:, 512:])

    xs2[...] = jnp.zeros_like(xs2)
    xs2[:, 1:33, :] = p1.reshape(BLK, 32, 512).astype(jnp.bfloat16)
    p2 = _conv_pool(xs2, wb2_ref, be2_ref, BLK=BLK, H=32, L=512, Nh=512)

    xs3[...] = jnp.zeros_like(xs3)
    xs3[:, 1:17, :] = p2.reshape(BLK, 16, 512).astype(jnp.bfloat16)
    p3 = _conv_pool(xs3, wb3_ref, be3_ref, BLK=BLK, H=16, L=512, Nh=512)

    r = p3.reshape(BLK, 8, 512)
    l0 = jnp.sum(jnp.sum(r * wf_ref[0], axis=2), axis=1, keepdims=True)
    l1 = jnp.sum(jnp.sum(r * wf_ref[1], axis=2), axis=1, keepdims=True)
    lane = jax.lax.broadcasted_iota(jnp.int32, (BLK, 2), 1)
    o_ref[...] = jnp.where(lane == 0, l0, l1) + bfc_ref[...]


def kernel(x, w1, b1, w2, b2, w3, b3, wfc, bfc):
    B = x.shape[0]
    BLK = _BLK
    wb1 = _band1(w1)
    wb2 = _bandN(w2, _J2, 32)
    wb3 = _bandN(w3, _J3, 16)
    be1 = _bias_ext(b1, 32)
    be2 = _bias_ext(b2, 16)
    be3 = _bias_ext(b3, 8)
    wf = wfc.reshape(64, 8, 8, 2).transpose(3, 1, 0, 2).reshape(2, 8, 512)
    wf = wf.astype(jnp.float32)
    bfc_p = bfc.reshape(1, 2).astype(jnp.float32)

    kernel_fn = functools.partial(_fused_kernel, BLK=BLK)
    out = pl.pallas_call(
        kernel_fn,
        out_shape=jax.ShapeDtypeStruct((B, 2), jnp.float32),
        grid=(B // BLK,),
        in_specs=[
            pl.BlockSpec((BLK, 3, 64, 64), lambda i: (i, 0, 0, 0)),
            pl.BlockSpec((576, 1024), lambda i: (0, 0)),
            pl.BlockSpec((1, 1024), lambda i: (0, 0)),
            pl.BlockSpec((1536, 1024), lambda i: (0, 0)),
            pl.BlockSpec((1, 1024), lambda i: (0, 0)),
            pl.BlockSpec((1536, 1024), lambda i: (0, 0)),
            pl.BlockSpec((1, 1024), lambda i: (0, 0)),
            pl.BlockSpec((2, 8, 512), lambda i: (0, 0, 0)),
            pl.BlockSpec((1, 2), lambda i: (0, 0)),
        ],
        out_specs=pl.BlockSpec((BLK, 2), lambda i: (i, 0)),
        scratch_shapes=[
            pltpu.VMEM((BLK, 3, 66, 64), jnp.bfloat16),
            pltpu.VMEM((BLK, 34, 512), jnp.bfloat16),
            pltpu.VMEM((BLK, 18, 512), jnp.bfloat16),
        ],
        compiler_params=pltpu.CompilerParams(
            dimension_semantics=("parallel",),
            vmem_limit_bytes=_VMEM_LIMIT),
    )(x, wb1, be1, wb2, be2, wb3, be3, wf, bfc_p)
    return out
